# Initial kernel scaffold; baseline (speedup 1.0000x reference)
import functools

import jax
import jax.numpy as jnp
from jax import lax
from jax.experimental import pallas as pl
from jax.experimental.pallas import tpu as pltpu
from jax.experimental.pallas import tpu_sc as plsc

B, F, L, D = 1024, 26, 50, 32
NW = 32
BLOCKS = B * F
BLK_PER_CHUNK = 8
IDX_PER_CHUNK = BLK_PER_CHUNK * L
OUT_PER_CHUNK = BLK_PER_CHUNK * L * D
CHUNKS = BLOCKS // BLK_PER_CHUNK
CPW = CHUNKS // NW
IDX_ROW = 100
IDX_SUB = IDX_PER_CHUNK // IDX_ROW


def _sc_body(idx_hbm, table_hbm, out_hbm, idx_v, rows_v, out_v, sem):
    wid = lax.axis_index("s") * 2 + lax.axis_index("c")
    iota50 = lax.iota(jnp.int32, 16) * 50

    def chunk_body(t, carry):
        c = wid * CPW + t
        pltpu.sync_copy(idx_hbm.at[c], idx_v)
        cps = [
            pltpu.async_copy(
                table_hbm.at[idx_v.at[j]],
                rows_v.at[pl.ds(j * IDX_ROW, IDX_ROW)],
                sem,
            )
            for j in range(IDX_SUB)
        ]
        for cp in cps:
            cp.wait()

        def g_body(g, carry2):
            def l_body(l, carry3):
                r = g * L + l
                obase = g * (L * D) + l
                for d0 in (0, 16):
                    vals = rows_v[r, pl.ds(d0, 16)]
                    plsc.store_scatter(out_v, [iota50 + (obase + d0 * 50)], vals)
                return carry3

            return lax.fori_loop(0, L, l_body, carry2)

        lax.fori_loop(0, BLK_PER_CHUNK, g_body, 0)
        pltpu.sync_copy(out_v, out_hbm.at[c])
        return carry

    lax.fori_loop(0, CPW, chunk_body, 0)


def kernel(inputs, table):
    assert inputs.shape == (B, F, L) and table.shape[1] == D
    idx = inputs.reshape(CHUNKS, IDX_SUB, IDX_ROW).astype(jnp.int32)

    mesh = plsc.VectorSubcoreMesh(core_axis_name="c", subcore_axis_name="s")
    out = pl.kernel(
        _sc_body,
        out_type=jax.ShapeDtypeStruct((CHUNKS, OUT_PER_CHUNK), jnp.float32),
        mesh=mesh,
        scratch_types=[
            pltpu.VMEM((IDX_SUB, IDX_ROW), jnp.int32),
            pltpu.VMEM((IDX_PER_CHUNK, D), jnp.float32),
            pltpu.VMEM((OUT_PER_CHUNK,), jnp.float32),
            pltpu.SemaphoreType.DMA,
        ],
    )(idx, table)
    return out.reshape(B, F, D, L)


# speedup vs baseline: 2.8466x; 2.8466x over previous
import functools

import jax
import jax.numpy as jnp
from jax import lax
from jax.experimental import pallas as pl
from jax.experimental.pallas import tpu as pltpu
from jax.experimental.pallas import tpu_sc as plsc

B, F, L, D = 1024, 26, 50, 32
NW = 32
BLOCKS = B * F
BLK_PER_CHUNK = 8
IDX_PER_CHUNK = BLK_PER_CHUNK * L
OUT_PER_CHUNK = BLK_PER_CHUNK * L * D
CHUNKS = BLOCKS // BLK_PER_CHUNK
CPW = CHUNKS // NW
IDX_ROW = 100
IDX_SUB = IDX_PER_CHUNK // IDX_ROW


def _sc_body(idx_hbm, table_hbm, out_hbm, idx_v, rows_v, out_v, sem):
    wid = lax.axis_index("s") * 2 + lax.axis_index("c")
    iota50 = lax.iota(jnp.int32, 16) * 50

    def chunk_body(t, carry):
        c = wid * CPW + t
        pltpu.sync_copy(idx_hbm.at[c], idx_v)
        cps = [
            pltpu.async_copy(
                table_hbm.at[idx_v.at[j]],
                rows_v.at[pl.ds(j * IDX_ROW, IDX_ROW)],
                sem,
            )
            for j in range(IDX_SUB)
        ]
        for cp in cps:
            cp.wait()

        def g_body(g, carry2):
            def l_body(l, carry3):
                r = g * L + l
                obase = g * (L * D) + l
                for d0 in (0, 16):
                    vals = rows_v[r, pl.ds(d0, 16)]
                    plsc.store_scatter(out_v, [iota50 + (obase + d0 * 50)], vals)
                return carry3

            return lax.fori_loop(0, L, l_body, carry2)

        lax.fori_loop(0, BLK_PER_CHUNK, g_body, 0)
        pltpu.sync_copy(out_v, out_hbm.at[c])
        return carry

    lax.fori_loop(0, CPW, chunk_body, 0)


def kernel(inputs, table):
    assert inputs.shape == (B, F, L) and table.shape[1] == D
    idx = inputs.reshape(CHUNKS, IDX_SUB, IDX_ROW).astype(jnp.int32)

    mesh = plsc.VectorSubcoreMesh(core_axis_name="c", subcore_axis_name="s")
    out = pl.kernel(
        _sc_body,
        out_type=jax.ShapeDtypeStruct((CHUNKS, OUT_PER_CHUNK), jnp.float32),
        mesh=mesh,
        compiler_params=pltpu.CompilerParams(
            needs_layout_passes=False, use_tc_tiling_on_sc=False
        ),
        scratch_types=[
            pltpu.VMEM((IDX_SUB, IDX_ROW), jnp.int32),
            pltpu.VMEM((IDX_PER_CHUNK, D), jnp.float32),
            pltpu.VMEM((OUT_PER_CHUNK,), jnp.float32),
            pltpu.SemaphoreType.DMA,
        ],
    )(idx, table)
    return out.reshape(B, F, D, L)


# speedup vs baseline: 3.7870x; 1.3304x over previous
import functools

import jax
import jax.numpy as jnp
from jax import lax
from jax.experimental import pallas as pl
from jax.experimental.pallas import tpu as pltpu
from jax.experimental.pallas import tpu_sc as plsc

B, F, L, D = 1024, 26, 50, 32
NW = 32
BLOCKS = B * F
BLK_PER_CHUNK = 8
IDX_PER_CHUNK = BLK_PER_CHUNK * L
OUT_PER_CHUNK = BLK_PER_CHUNK * L * D
CHUNKS = BLOCKS // BLK_PER_CHUNK
CPW = CHUNKS // NW
IDX_ROW = 100
IDX_SUB = IDX_PER_CHUNK // IDX_ROW
BLK_OUT = L * D
SCAT_WIN = 1551


def _sc_body(idx_hbm, table_hbm, out_hbm, idx_v, rows_v, out_v,
             gs0, gs1, os0, os1):
    wid = lax.axis_index("s") * 2 + lax.axis_index("c")
    c0 = wid * CPW
    gsems = (gs0, gs1)
    osems = (os0, os1)
    i50 = lax.iota(jnp.int32, 16) * 50
    i50b = i50 + 16 * 50

    def issue(cl, slot):
        pltpu.sync_copy(idx_hbm.at[c0 + cl], idx_v.at[slot])
        for j in range(IDX_SUB):
            pltpu.async_copy(
                table_hbm.at[idx_v.at[slot].at[j]],
                rows_v.at[slot].at[pl.ds(j * IDX_ROW, IDX_ROW)],
                gsems[slot],
            )

    def wait_gather(slot):
        pltpu.make_async_copy(
            table_hbm.at[pl.ds(0, IDX_PER_CHUNK)], rows_v.at[slot], gsems[slot]
        ).wait()

    def transpose(slot):
        rows = rows_v.at[slot]
        outs = out_v.at[slot]

        @plsc.parallel_loop(0, L, unroll=2)
        def _(l):
            dv0 = i50 + l
            dv1 = i50b + l
            for g in range(BLK_PER_CHUNK):
                sl = outs.at[pl.ds(g * BLK_OUT, BLK_OUT)]
                r = g * L + l
                plsc.store_scatter(sl, [dv0], rows[r, pl.ds(0, 16)])
                plsc.store_scatter(sl, [dv1], rows[r, pl.ds(16, 16)])

    issue(0, 0)

    @pl.loop(0, CPW, step=2)
    def _(t):
        for b in range(2):
            cl = t + b

            @pl.when(cl + 1 < CPW)
            def _():
                issue(cl + 1, 1 - b)

            wait_gather(b)

            @pl.when(cl >= 2)
            def _():
                pltpu.make_async_copy(
                    out_v.at[b], out_hbm.at[c0 + cl - 2], osems[b]
                ).wait()

            transpose(b)
            pltpu.async_copy(out_v.at[b], out_hbm.at[c0 + cl], osems[b])

    pltpu.make_async_copy(out_v.at[0], out_hbm.at[c0 + CPW - 2], osems[0]).wait()
    pltpu.make_async_copy(out_v.at[1], out_hbm.at[c0 + CPW - 1], osems[1]).wait()


def kernel(inputs, table):
    assert inputs.shape == (B, F, L) and table.shape[1] == D
    idx = inputs.reshape(CHUNKS, IDX_SUB, IDX_ROW).astype(jnp.int32)

    mesh = plsc.VectorSubcoreMesh(core_axis_name="c", subcore_axis_name="s")
    out = pl.kernel(
        _sc_body,
        out_type=jax.ShapeDtypeStruct((CHUNKS, OUT_PER_CHUNK), jnp.float32),
        mesh=mesh,
        compiler_params=pltpu.CompilerParams(
            needs_layout_passes=False, use_tc_tiling_on_sc=False
        ),
        scratch_types=[
            pltpu.VMEM((2, IDX_SUB, IDX_ROW), jnp.int32),
            pltpu.VMEM((2, IDX_PER_CHUNK, D), jnp.float32),
            pltpu.VMEM((2, OUT_PER_CHUNK), jnp.float32),
            pltpu.SemaphoreType.DMA,
            pltpu.SemaphoreType.DMA,
            pltpu.SemaphoreType.DMA,
            pltpu.SemaphoreType.DMA,
        ],
    )(idx, table)
    return out.reshape(B, F, D, L)


# speedup vs baseline: 4.4590x; 1.1774x over previous
import jax
import jax.numpy as jnp
from jax import lax
from jax.experimental import pallas as pl
from jax.experimental.pallas import tpu as pltpu
from jax.experimental.pallas import tpu_sc as plsc

B, F, L, D = 1024, 26, 50, 32
NW = 32
HALF = F // 2
IDX_PER_CHUNK = HALF * L
CHUNKS = B * 2
CPW = CHUNKS // NW


def _sc_body(idx_hbm, table_hbm, out_hbm, idx_v, rows_v, out_v,
             gs0, gs1, os0, os1):
    wid = lax.axis_index("s") * 2 + lax.axis_index("c")
    c0 = wid * CPW
    gsems = (gs0, gs1)
    osems = (os0, os1)
    iota = lax.iota(jnp.int32, 16)
    dvs = (iota, iota + 16)

    def issue(cl, slot):
        ch = c0 + cl
        b = ch // 2
        f0 = (ch % 2) * HALF
        pltpu.sync_copy(idx_hbm.at[b].at[pl.ds(f0, HALF)], idx_v.at[slot])
        for j in range(HALF):
            pltpu.async_copy(
                table_hbm.at[idx_v.at[slot].at[j]],
                rows_v.at[slot].at[pl.ds(j * L, L)],
                gsems[slot],
            )

    def wait_gather(slot):
        pltpu.make_async_copy(
            table_hbm.at[pl.ds(0, IDX_PER_CHUNK)], rows_v.at[slot], gsems[slot]
        ).wait()

    def out_view(cl):
        ch = c0 + cl
        return out_hbm.at[ch // 2].at[pl.ds((ch % 2) * HALF, HALF)]

    def transpose(slot):
        rows = rows_v.at[slot]
        outs = out_v.at[slot]

        @plsc.parallel_loop(0, L, unroll=2)
        def _(l):
            lv = iota * 0 + l
            for g in range(HALF):
                blk = outs.at[g]
                r = g * L + l
                plsc.store_scatter(blk, [dvs[0], lv], rows[r, pl.ds(0, 16)])
                plsc.store_scatter(blk, [dvs[1], lv], rows[r, pl.ds(16, 16)])

    issue(0, 0)

    @pl.loop(0, CPW, step=2)
    def _(t):
        for b in range(2):
            cl = t + b

            @pl.when(cl + 1 < CPW)
            def _():
                issue(cl + 1, 1 - b)

            wait_gather(b)

            @pl.when(cl >= 2)
            def _():
                pltpu.make_async_copy(
                    out_v.at[b], out_view(cl - 2), osems[b]
                ).wait()

            transpose(b)
            pltpu.async_copy(out_v.at[b], out_view(cl), osems[b])

    pltpu.make_async_copy(out_v.at[0], out_view(CPW - 2), osems[0]).wait()
    pltpu.make_async_copy(out_v.at[1], out_view(CPW - 1), osems[1]).wait()


def kernel(inputs, table):
    assert inputs.shape == (B, F, L) and table.shape[1] == D
    idx = inputs.astype(jnp.int32)

    mesh = plsc.VectorSubcoreMesh(core_axis_name="c", subcore_axis_name="s")
    out = pl.kernel(
        _sc_body,
        out_type=jax.ShapeDtypeStruct((B, F, D, L), jnp.float32),
        mesh=mesh,
        compiler_params=pltpu.CompilerParams(
            needs_layout_passes=False, use_tc_tiling_on_sc=False
        ),
        scratch_types=[
            pltpu.VMEM((2, HALF, L), jnp.int32),
            pltpu.VMEM((2, IDX_PER_CHUNK, D), jnp.float32),
            pltpu.VMEM((2, HALF, D, L), jnp.float32),
            pltpu.SemaphoreType.DMA,
            pltpu.SemaphoreType.DMA,
            pltpu.SemaphoreType.DMA,
            pltpu.SemaphoreType.DMA,
        ],
    )(idx, table)
    return out


# speedup vs baseline: 4.8881x; 1.0962x over previous
import jax
import jax.numpy as jnp
from jax import lax
from jax.experimental import pallas as pl
from jax.experimental.pallas import tpu as pltpu
from jax.experimental.pallas import tpu_sc as plsc

B, F, L, D = 1024, 26, 50, 32
NW = 32
BT = B // 128
LSTEP = 5
NLQ = L // LSTEP
STAGES = F * BT * NLQ
SPW = STAGES // NW
NR = 128 * LSTEP


def _sc_body(idx_hbm, table_hbm, out_hbm, idxr_v, idxt_v, rows_v, out_v,
             gs0, gs1, os0, os1, is0, is1):
    wid = lax.axis_index("s") * 2 + lax.axis_index("c")
    s0 = wid * SPW
    gsems = (gs0, gs1)
    osems = (os0, os1)
    isems = (is0, is1)
    iota = lax.iota(jnp.int32, 16)

    def decode(st):
        lq = st % NLQ
        fb = st // NLQ
        return fb // BT, fb % BT, lq

    def issue_idx(st, slot):
        f, bh, lq = decode(st)
        pltpu.async_copy(
            idx_hbm.at[pl.ds(bh * 128, 128), f],
            idxr_v.at[slot],
            isems[slot],
        )

    def wait_idx(slot):
        pltpu.make_async_copy(
            idx_hbm.at[pl.ds(0, 128), 0],
            idxr_v.at[slot],
            isems[slot],
        ).wait()

    def issue_gather(st, slot):
        _, _, lq = decode(st)
        raw = idxr_v.at[slot]
        tr = idxt_v.at[slot]
        for c in range(LSTEP):
            cv = iota * 0 + (lq * LSTEP + c)
            for r0 in range(0, 128, 16):
                v = plsc.load_gather(raw, [iota + r0, cv])
                tr[c, pl.ds(r0, 16)] = v
        for c in range(LSTEP):
            pltpu.async_copy(
                table_hbm.at[tr.at[c]],
                rows_v.at[slot].at[pl.ds(c * 128, 128)],
                gsems[slot],
            )

    def wait_gather(slot):
        pltpu.make_async_copy(
            table_hbm.at[pl.ds(0, NR)],
            rows_v.at[slot],
            gsems[slot],
        ).wait()

    def out_view(st):
        f, bh, lq = decode(st)
        return out_hbm.at[f, pl.ds(lq * LSTEP, LSTEP), :, bh]

    def shuffle(slot):
        rows = rows_v.at[slot]
        outs = out_v.at[slot]

        @plsc.parallel_loop(0, LSTEP * D, unroll=2)
        def _(i):
            lrel = i // D
            d = i - lrel * D
            rv = iota + lrel * 128
            cv = iota * 0 + d
            for k in range(8):
                v = plsc.load_gather(rows, [rv + k * 16, cv])
                outs[lrel, d // 8, d % 8, pl.ds(k * 16, 16)] = v

    issue_idx(s0, 0)
    wait_idx(0)
    issue_gather(s0, 0)

    @pl.loop(0, SPW + 1, step=2)
    def _(t):
        for b in range(2):
            st = t + b

            @pl.when(st + 1 < SPW)
            def _():
                issue_idx(s0 + st + 1, 1 - b)
                wait_idx(1 - b)
                issue_gather(s0 + st + 1, 1 - b)

            @pl.when(st < SPW)
            def _():
                wait_gather(b)

                @pl.when(st >= 2)
                def _():
                    pltpu.make_async_copy(
                        out_v.at[b], out_view(s0 + st - 2), osems[b]
                    ).wait()

                shuffle(b)
                pltpu.async_copy(out_v.at[b], out_view(s0 + st), osems[b])

    sl0 = (SPW - 2) % 2
    sl1 = (SPW - 1) % 2
    pltpu.make_async_copy(out_v.at[sl0], out_view(s0 + SPW - 2), osems[sl0]).wait()
    pltpu.make_async_copy(out_v.at[sl1], out_view(s0 + SPW - 1), osems[sl1]).wait()


def kernel(inputs, table):
    assert inputs.shape == (B, F, L) and table.shape[1] == D
    idx = inputs.astype(jnp.int32)

    mesh = plsc.VectorSubcoreMesh(core_axis_name="c", subcore_axis_name="s")
    out6 = pl.kernel(
        _sc_body,
        out_type=jax.ShapeDtypeStruct((F, L, 4, 8, 8, 128), jnp.float32),
        mesh=mesh,
        compiler_params=pltpu.CompilerParams(
            needs_layout_passes=False, use_tc_tiling_on_sc=False
        ),
        scratch_types=[
            pltpu.VMEM((2, 128, L), jnp.int32),
            pltpu.VMEM((2, LSTEP, 128), jnp.int32),
            pltpu.VMEM((2, NR, D), jnp.float32),
            pltpu.VMEM((2, LSTEP, 4, 8, 128), jnp.float32),
            pltpu.SemaphoreType.DMA,
            pltpu.SemaphoreType.DMA,
            pltpu.SemaphoreType.DMA,
            pltpu.SemaphoreType.DMA,
            pltpu.SemaphoreType.DMA,
            pltpu.SemaphoreType.DMA,
        ],
    )(idx, table)
    return out6.transpose(3, 5, 0, 2, 4, 1).reshape(B, F, D, L)


# speedup vs baseline: 7.3273x; 1.4990x over previous
import jax
import jax.numpy as jnp
from jax import lax
from jax.experimental import pallas as pl
from jax.experimental.pallas import tpu as pltpu
from jax.experimental.pallas import tpu_sc as plsc

B, F, L, D = 1024, 26, 50, 32
NW = 32
BT = B // 128
LSTEP = 5
NLQ = L // LSTEP
STAGES = F * BT * NLQ
SPW = STAGES // NW
NR = 128 * LSTEP
RP = D + 1


def _sc_body(idx_hbm, table_hbm, out_hbm, idxr_v, idxt_v, rows_v, rowsp_v,
             out_v, gs0, gs1, os0, os1, isem):
    wid = lax.axis_index("s") * 2 + lax.axis_index("c")
    s0 = wid * SPW
    gsems = (gs0, gs1)
    osems = (os0, os1)
    iota = lax.iota(jnp.int32, 16)
    iota_rp = iota * RP

    def decode(st):
        lq = st % NLQ
        fb = st // NLQ
        return fb // BT, fb % BT, lq

    def issue_idx(st):
        f, bh, lq = decode(st)
        pltpu.async_copy(
            idx_hbm.at[pl.ds(bh * 128, 128), f],
            idxr_v,
            isem,
        )

    def wait_idx():
        pltpu.make_async_copy(
            idx_hbm.at[pl.ds(0, 128), 0],
            idxr_v,
            isem,
        ).wait()

    def issue_gather(st, slot):
        _, _, lq = decode(st)
        raw = idxr_v
        tr = idxt_v.at[slot]
        for c in range(LSTEP):
            cv = iota * 0 + (lq * LSTEP + c)
            for r0 in range(0, 128, 16):
                v = plsc.load_gather(raw, [iota + r0, cv])
                tr[c, pl.ds(r0, 16)] = v
        for c in range(LSTEP):
            pltpu.async_copy(
                table_hbm.at[tr.at[c]],
                rows_v.at[slot].at[pl.ds(c * 128, 128)],
                gsems[slot],
            )

    def wait_gather(slot):
        pltpu.make_async_copy(
            table_hbm.at[pl.ds(0, NR)],
            rows_v.at[slot],
            gsems[slot],
        ).wait()

    def out_view(st):
        f, bh, lq = decode(st)
        return out_hbm.at[f, pl.ds(lq * LSTEP, LSTEP), :, bh]

    def shuffle(slot):
        rows = rows_v.at[slot]
        outs = out_v.at[slot]

        @plsc.parallel_loop(0, NR, unroll=2)
        def _(r):
            base = r * RP
            for d0 in (0, 16):
                sv = iota + (base + d0)
                plsc.store_scatter(rowsp_v, [sv], rows[r, pl.ds(d0, 16)])

        @plsc.parallel_loop(0, LSTEP * D, unroll=2)
        def _(i):
            lrel = i // D
            d = i - lrel * D
            for k in range(8):
                gv = iota_rp + ((lrel * 128 + k * 16) * RP + d)
                v = plsc.load_gather(rowsp_v, [gv])
                outs[lrel, d // 8, d % 8, pl.ds(k * 16, 16)] = v

    issue_idx(s0)
    wait_idx()
    issue_gather(s0, 0)

    @pl.loop(0, SPW + 1, step=2)
    def _(t):
        for b in range(2):
            st = t + b

            @pl.when(st + 1 < SPW)
            def _():
                issue_idx(s0 + st + 1)
                wait_idx()
                issue_gather(s0 + st + 1, 1 - b)

            @pl.when(st < SPW)
            def _():
                wait_gather(b)

                @pl.when(st >= 2)
                def _():
                    pltpu.make_async_copy(
                        out_v.at[b], out_view(s0 + st - 2), osems[b]
                    ).wait()

                shuffle(b)
                pltpu.async_copy(out_v.at[b], out_view(s0 + st), osems[b])

    sl0 = (SPW - 2) % 2
    sl1 = (SPW - 1) % 2
    pltpu.make_async_copy(out_v.at[sl0], out_view(s0 + SPW - 2), osems[sl0]).wait()
    pltpu.make_async_copy(out_v.at[sl1], out_view(s0 + SPW - 1), osems[sl1]).wait()


def kernel(inputs, table):
    assert inputs.shape == (B, F, L) and table.shape[1] == D
    idx = inputs.astype(jnp.int32)

    mesh = plsc.VectorSubcoreMesh(core_axis_name="c", subcore_axis_name="s")
    out6 = pl.kernel(
        _sc_body,
        out_type=jax.ShapeDtypeStruct((F, L, 4, 8, 8, 128), jnp.float32),
        mesh=mesh,
        compiler_params=pltpu.CompilerParams(
            needs_layout_passes=False, use_tc_tiling_on_sc=False
        ),
        scratch_types=[
            pltpu.VMEM((128, L), jnp.int32),
            pltpu.VMEM((2, LSTEP, 128), jnp.int32),
            pltpu.VMEM((2, NR, D), jnp.float32),
            pltpu.VMEM((NR * RP,), jnp.float32),
            pltpu.VMEM((2, LSTEP, 4, 8, 128), jnp.float32),
            pltpu.SemaphoreType.DMA,
            pltpu.SemaphoreType.DMA,
            pltpu.SemaphoreType.DMA,
            pltpu.SemaphoreType.DMA,
            pltpu.SemaphoreType.DMA,
        ],
    )(idx, table)
    return out6.transpose(3, 5, 0, 2, 4, 1).reshape(B, F, D, L)
